# Initial kernel scaffold; baseline (speedup 1.0000x reference)
import jax
import jax.numpy as jnp
from jax.experimental import pallas as pl


def kernel(emb_e, W1, W2, W3, re_attention_weight, u, en_weight, re_weight, re_specific_attention, edge_index, edge_type):
    raise NotImplementedError("write your pallas kernel here")


# speedup vs baseline: 3.2768x; 3.2768x over previous
import functools

import jax
import jax.numpy as jnp
from jax import lax
from jax.experimental import pallas as pl
from jax.experimental.pallas import tpu as pltpu
from jax.experimental.pallas import tpu_sc as plsc

N = 10000
E = 320000
D = 128
R = 16
NC = 2
NS = 16
L = 16
NW = NC * NS
EPW = E // NW
C = 80
NCH = EPW // C
NPAD = 10240
RPW = NPAD // NS
RB = 128

_mesh = lambda: plsc.VectorSubcoreMesh(
    core_axis_name="c", subcore_axis_name="s", num_cores=NC, num_subcores=NS)


def _zero_vmem_rows(buf, nrows):
  def zb(i, carry):
    for j in range(D // L):
      buf[i, pl.ds(j * L, L)] = jnp.zeros((L,), jnp.float32)
    return carry
  lax.fori_loop(0, nrows, zb, 0)


def _zero_vmem_1d(buf, n):
  def zd(i, carry):
    buf[pl.ds(i * L, L)] = jnp.zeros((L,), jnp.float32)
    return carry
  lax.fori_loop(0, n // L, zd, 0)


def _scale_rows(rows, wch, n):
  def sc(i, carry):
    w16 = plsc.load_gather(wch, [jnp.full((L,), i, jnp.int32)])
    for j in range(D // L):
      rows[i, pl.ds(j * L, L)] = rows[i, pl.ds(j * L, L)] * w16
    return carry
  lax.fori_loop(0, n, sc, 0)


def _edges_first(x, src, dst, et, wtab):
  @functools.partial(
      pl.kernel,
      out_type=(
          jax.ShapeDtypeStruct((NC, NPAD, D), jnp.float32),
          jax.ShapeDtypeStruct((NC, NPAD), jnp.float32),
          jax.ShapeDtypeStruct((E,), jnp.float32),
      ),
      mesh=_mesh(),
      compiler_params=pltpu.CompilerParams(needs_layout_passes=False),
      scratch_types=[
          pltpu.VMEM((RB, D), jnp.float32),
          pltpu.VMEM((C,), jnp.int32),
          pltpu.VMEM((C,), jnp.int32),
          pltpu.VMEM((C,), jnp.int32),
          pltpu.VMEM((D,), jnp.float32),
          pltpu.VMEM((C, D), jnp.float32),
          pltpu.VMEM((D,), jnp.float32),
          pltpu.VMEM((RPW,), jnp.float32),
          pltpu.VMEM_SHARED((NPAD, D), jnp.float32),
          pltpu.VMEM_SHARED((NPAD,), jnp.float32),
          pltpu.SemaphoreType.DMA,
      ],
  )
  def k(x_hbm, src_hbm, dst_hbm, et_hbm, wtab_hbm, aggp, degp, weo,
        zbuf, idx_s, idx_d, etv, wch, rows, wtabv, zdeg, agg_sh, deg_sh, sem):
    c = lax.axis_index("c")
    s = lax.axis_index("s")
    wid = c * NS + s
    _zero_vmem_rows(zbuf, RB)
    _zero_vmem_1d(zdeg, RPW)
    for kk in range(RPW // RB):
      pltpu.sync_copy(zbuf, agg_sh.at[pl.ds(s * RPW + kk * RB, RB)])
    pltpu.sync_copy(zdeg, deg_sh.at[pl.ds(s * RPW, RPW)])
    pltpu.sync_copy(wtab_hbm, wtabv)
    plsc.subcore_barrier()

    base = wid * EPW

    def chunk(g, carry):
      off = base + g * C
      pltpu.sync_copy(src_hbm.at[pl.ds(off, C)], idx_s)
      pltpu.sync_copy(dst_hbm.at[pl.ds(off, C)], idx_d)
      pltpu.sync_copy(et_hbm.at[pl.ds(off, C)], etv)
      pltpu.async_copy(x_hbm.at[idx_s], rows, sem).wait()

      def wg(kk, cy):
        ev = etv[pl.ds(kk * L, L)]
        wch[pl.ds(kk * L, L)] = plsc.load_gather(wtabv, [ev])
        return cy
      lax.fori_loop(0, C // L, wg, 0)

      _scale_rows(rows, wch, C)
      pltpu.sync_copy(wch.at[pl.ds(0, C)], weo.at[pl.ds(off, C)])
      pltpu.sync_copy(rows, agg_sh.at[idx_d], add=True)
      pltpu.sync_copy(wch.at[pl.ds(0, C)], deg_sh.at[idx_d], add=True)
      return carry

    lax.fori_loop(0, NCH, chunk, 0)
    plsc.subcore_barrier()
    pltpu.sync_copy(agg_sh.at[pl.ds(s * RPW, RPW)],
                    aggp.at[c, pl.ds(s * RPW, RPW)])
    pltpu.sync_copy(deg_sh.at[pl.ds(s * RPW, RPW)],
                    degp.at[c, pl.ds(s * RPW, RPW)])

  return k(x, src, dst, et, wtab)


def _edges_next(x, src, dst, we):
  @functools.partial(
      pl.kernel,
      out_type=jax.ShapeDtypeStruct((NC, NPAD, D), jnp.float32),
      mesh=_mesh(),
      compiler_params=pltpu.CompilerParams(needs_layout_passes=False),
      scratch_types=[
          pltpu.VMEM((RB, D), jnp.float32),
          pltpu.VMEM((C,), jnp.int32),
          pltpu.VMEM((C,), jnp.int32),
          pltpu.VMEM((D,), jnp.float32),
          pltpu.VMEM((C, D), jnp.float32),
          pltpu.VMEM_SHARED((NPAD, D), jnp.float32),
          pltpu.SemaphoreType.DMA,
      ],
  )
  def k(x_hbm, src_hbm, dst_hbm, we_hbm, aggp,
        zbuf, idx_s, idx_d, wch, rows, agg_sh, sem):
    c = lax.axis_index("c")
    s = lax.axis_index("s")
    wid = c * NS + s
    _zero_vmem_rows(zbuf, RB)
    for kk in range(RPW // RB):
      pltpu.sync_copy(zbuf, agg_sh.at[pl.ds(s * RPW + kk * RB, RB)])
    plsc.subcore_barrier()

    base = wid * EPW

    def chunk(g, carry):
      off = base + g * C
      pltpu.sync_copy(src_hbm.at[pl.ds(off, C)], idx_s)
      pltpu.sync_copy(dst_hbm.at[pl.ds(off, C)], idx_d)
      pltpu.sync_copy(we_hbm.at[pl.ds(off, C)], wch.at[pl.ds(0, C)])
      pltpu.async_copy(x_hbm.at[idx_s], rows, sem).wait()
      _scale_rows(rows, wch, C)
      pltpu.sync_copy(rows, agg_sh.at[idx_d], add=True)
      return carry

    lax.fori_loop(0, NCH, chunk, 0)
    plsc.subcore_barrier()
    pltpu.sync_copy(agg_sh.at[pl.ds(s * RPW, RPW)],
                    aggp.at[c, pl.ds(s * RPW, RPW)])

  return k(x, src, dst, we)


def _attn_logits(sa, sb, src, dst, cvec):
  @functools.partial(
      pl.kernel,
      out_type=(
          jax.ShapeDtypeStruct((E,), jnp.float32),
          jax.ShapeDtypeStruct((NW, L), jnp.float32),
      ),
      mesh=_mesh(),
      compiler_params=pltpu.CompilerParams(needs_layout_passes=False),
      scratch_types=[
          pltpu.VMEM((NPAD,), jnp.float32),
          pltpu.VMEM((NPAD,), jnp.float32),
          pltpu.VMEM((EPW,), jnp.int32),
          pltpu.VMEM((EPW,), jnp.int32),
          pltpu.VMEM((EPW,), jnp.float32),
          pltpu.VMEM((L,), jnp.float32),
          pltpu.VMEM((L,), jnp.float32),
      ],
  )
  def k(sa_hbm, sb_hbm, src_hbm, dst_hbm, c_hbm, e_out, mx_out,
        sav, sbv, srcv, dstv, ev, mv, cv):
    c = lax.axis_index("c")
    s = lax.axis_index("s")
    wid = c * NS + s
    base = wid * EPW
    pltpu.sync_copy(sa_hbm, sav.at[pl.ds(0, N)])
    pltpu.sync_copy(sb_hbm, sbv.at[pl.ds(0, N)])
    pltpu.sync_copy(src_hbm.at[pl.ds(base, EPW)], srcv)
    pltpu.sync_copy(dst_hbm.at[pl.ds(base, EPW)], dstv)
    pltpu.sync_copy(c_hbm, cv)
    c16 = cv[...]

    def step(kk, m):
      sv = srcv[pl.ds(kk * L, L)]
      dv = dstv[pl.ds(kk * L, L)]
      a16 = plsc.load_gather(sav, [sv])
      b16 = plsc.load_gather(sbv, [dv])
      e16 = a16 + b16 + c16
      e16 = jnp.where(e16 >= 0.0, e16, e16 * jnp.float32(0.01))
      ev[pl.ds(kk * L, L)] = e16
      return jnp.maximum(m, e16)

    m = lax.fori_loop(0, EPW // L, step,
                      jnp.full((L,), -jnp.inf, jnp.float32))
    mv[...] = m
    pltpu.sync_copy(ev, e_out.at[pl.ds(base, EPW)])
    pltpu.sync_copy(mv, mx_out.at[wid])

  return k(sa, sb, src, dst, cvec)


def _attn_exp(e, mx):
  @functools.partial(
      pl.kernel,
      out_type=(
          jax.ShapeDtypeStruct((E,), jnp.float32),
          jax.ShapeDtypeStruct((NW, L), jnp.float32),
      ),
      mesh=_mesh(),
      compiler_params=pltpu.CompilerParams(needs_layout_passes=False),
      scratch_types=[
          pltpu.VMEM((EPW,), jnp.float32),
          pltpu.VMEM((NW, L), jnp.float32),
          pltpu.VMEM((L,), jnp.float32),
      ],
  )
  def k(e_hbm, mx_hbm, ex_out, sm_out, ev, mxv, sv):
    c = lax.axis_index("c")
    s = lax.axis_index("s")
    wid = c * NS + s
    base = wid * EPW
    pltpu.sync_copy(mx_hbm, mxv)
    m = mxv[0, :]
    for w in range(1, NW):
      m = jnp.maximum(m, mxv[w, :])
    gm = jnp.max(m)
    pltpu.sync_copy(e_hbm.at[pl.ds(base, EPW)], ev)

    def step(kk, acc):
      x16 = jnp.exp(ev[pl.ds(kk * L, L)] - gm)
      ev[pl.ds(kk * L, L)] = x16
      return acc + x16

    acc = lax.fori_loop(0, EPW // L, step, jnp.zeros((L,), jnp.float32))
    sv[...] = acc
    pltpu.sync_copy(ev, ex_out.at[pl.ds(base, EPW)])
    pltpu.sync_copy(sv, sm_out.at[wid])

  return k(e, mx)


def _attn_scatter(x, src, dst, ex, sm):
  @functools.partial(
      pl.kernel,
      out_type=jax.ShapeDtypeStruct((NC, NPAD, D), jnp.float32),
      mesh=_mesh(),
      compiler_params=pltpu.CompilerParams(needs_layout_passes=False),
      scratch_types=[
          pltpu.VMEM((RB, D), jnp.float32),
          pltpu.VMEM((C,), jnp.int32),
          pltpu.VMEM((C,), jnp.int32),
          pltpu.VMEM((D,), jnp.float32),
          pltpu.VMEM((C, D), jnp.float32),
          pltpu.VMEM((NW, L), jnp.float32),
          pltpu.VMEM_SHARED((NPAD, D), jnp.float32),
          pltpu.SemaphoreType.DMA,
      ],
  )
  def k(x_hbm, src_hbm, dst_hbm, ex_hbm, sm_hbm, outp,
        zbuf, idx_s, idx_d, wch, rows, smv, out_sh, sem):
    c = lax.axis_index("c")
    s = lax.axis_index("s")
    wid = c * NS + s
    _zero_vmem_rows(zbuf, RB)
    for kk in range(RPW // RB):
      pltpu.sync_copy(zbuf, out_sh.at[pl.ds(s * RPW + kk * RB, RB)])
    pltpu.sync_copy(sm_hbm, smv)
    acc = smv[0, :]
    for w in range(1, NW):
      acc = acc + smv[w, :]
    total = jnp.sum(acc)
    inv16 = jnp.ones((L,), jnp.float32) / (
        jnp.full((L,), total, jnp.float32) + jnp.float32(1e-9))
    plsc.subcore_barrier()

    base = wid * EPW

    def chunk(g, carry):
      off = base + g * C
      pltpu.sync_copy(src_hbm.at[pl.ds(off, C)], idx_s)
      pltpu.sync_copy(dst_hbm.at[pl.ds(off, C)], idx_d)
      pltpu.sync_copy(ex_hbm.at[pl.ds(off, C)], wch.at[pl.ds(0, C)])

      def nr(kk, cy):
        wch[pl.ds(kk * L, L)] = wch[pl.ds(kk * L, L)] * inv16
        return cy
      lax.fori_loop(0, C // L, nr, 0)

      pltpu.async_copy(x_hbm.at[idx_d], rows, sem).wait()
      _scale_rows(rows, wch, C)
      pltpu.sync_copy(rows, out_sh.at[idx_s], add=True)
      return carry

    lax.fori_loop(0, NCH, chunk, 0)
    plsc.subcore_barrier()
    pltpu.sync_copy(out_sh.at[pl.ds(s * RPW, RPW)],
                    outp.at[c, pl.ds(s * RPW, RPW)])

  return k(x, src, dst, ex, sm)


BR = 80
NBLK = N // BR


def _prep_tc(raw, u3, en_w, re_w, rsa):
  def body(raw_ref, u3_ref, en_ref, rew_ref, rsa_ref,
           wtab_ref, av_ref, bv_ref, cv_ref):
    wtab_ref[...] = jax.nn.sigmoid(raw_ref[...])
    u0 = u3_ref[0, :][None, :]
    u1 = u3_ref[1, :]
    u2 = u3_ref[2, :][None, :]
    av_ref[...] = jnp.dot(u0, en_ref[...], preferred_element_type=jnp.float32)
    bv_ref[...] = jnp.dot(u2, en_ref[...], preferred_element_type=jnp.float32)
    r_term = jnp.sum(rew_ref[...] * rsa_ref[...], axis=1)
    cval = jnp.sum(r_term * u1)
    cv_ref[...] = jnp.full((1, L), cval, jnp.float32)

  return pl.pallas_call(
      body,
      out_shape=(
          jax.ShapeDtypeStruct((1, R), jnp.float32),
          jax.ShapeDtypeStruct((1, D), jnp.float32),
          jax.ShapeDtypeStruct((1, D), jnp.float32),
          jax.ShapeDtypeStruct((1, L), jnp.float32),
      ),
  )(raw, u3, en_w, re_w, rsa)


def _layer_tc(x, aggp, degp, W):
  def body(x_ref, agg_ref, deg_ref, w_ref, o_ref):
    agg = agg_ref[0] + agg_ref[1]
    deg = deg_ref[0] + deg_ref[1] + jnp.float32(1e-6)
    h = agg / deg + x_ref[...]
    o_ref[...] = jnp.maximum(
        jnp.dot(h, w_ref[...], preferred_element_type=jnp.float32), 0.0)

  return pl.pallas_call(
      body,
      grid=(NBLK,),
      in_specs=[
          pl.BlockSpec((BR, D), lambda g: (g, 0)),
          pl.BlockSpec((NC, BR, D), lambda g: (0, g, 0)),
          pl.BlockSpec((NC, BR, 1), lambda g: (0, g, 0)),
          pl.BlockSpec((D, D), lambda g: (0, 0)),
      ],
      out_specs=pl.BlockSpec((BR, D), lambda g: (g, 0)),
      out_shape=jax.ShapeDtypeStruct((N, D), jnp.float32),
  )(x, aggp, degp, W)


def _sab_tc(x, ab):
  def body(x_ref, ab_ref, o_ref):
    o_ref[...] = jnp.dot(x_ref[...], ab_ref[...],
                         preferred_element_type=jnp.float32)

  return pl.pallas_call(
      body,
      grid=(NBLK,),
      in_specs=[
          pl.BlockSpec((BR, D), lambda g: (g, 0)),
          pl.BlockSpec((D, 2), lambda g: (0, 0)),
      ],
      out_specs=pl.BlockSpec((BR, 2), lambda g: (g, 0)),
      out_shape=jax.ShapeDtypeStruct((N, 2), jnp.float32),
  )(x, ab)


def _combine_tc(outp):
  def body(p_ref, o_ref):
    o_ref[...] = p_ref[0] + p_ref[1]

  return pl.pallas_call(
      body,
      grid=(NBLK,),
      in_specs=[pl.BlockSpec((NC, BR, D), lambda g: (0, g, 0))],
      out_specs=pl.BlockSpec((BR, D), lambda g: (g, 0)),
      out_shape=jax.ShapeDtypeStruct((N, D), jnp.float32),
  )(outp)


def kernel(emb_e, W1, W2, W3, re_attention_weight, u, en_weight, re_weight,
           re_specific_attention, edge_index, edge_type):
  src = edge_index[0].astype(jnp.int32)
  dst = edge_index[1].astype(jnp.int32)
  et = edge_type.astype(jnp.int32)

  raw = re_attention_weight.reshape(1, R)
  u3 = u.reshape(3, D)
  rsa = re_specific_attention.reshape(1, D)

  wtab2, av, bv, cv = _prep_tc(raw, u3, en_weight, re_weight, rsa)
  wtab = jnp.pad(wtab2.reshape(R), (0, D - R))
  cvec = cv.reshape(L)

  aggp, degp, we = _edges_first(emb_e, src, dst, et, wtab)
  degp3 = degp.reshape(NC, NPAD, 1)
  x1 = _layer_tc(emb_e, aggp, degp3, W1)

  aggp2 = _edges_next(x1, src, dst, we)
  x2 = _layer_tc(x1, aggp2, degp3, W2)

  aggp3 = _edges_next(x2, src, dst, we)
  x3 = _layer_tc(x2, aggp3, degp3, W3)

  ab = jnp.stack([av[0], bv[0]], axis=1)
  sab = _sab_tc(x3, ab)
  sa = sab[:, 0]
  sb = sab[:, 1]

  e, mx = _attn_logits(sa, sb, src, dst, cvec)
  ex, sm = _attn_exp(e, mx)
  outp = _attn_scatter(x3, src, dst, ex, sm)
  return _combine_tc(outp)


# speedup vs baseline: 3.9782x; 1.2140x over previous
import functools

import jax
import jax.numpy as jnp
from jax import lax
from jax.experimental import pallas as pl
from jax.experimental.pallas import tpu as pltpu
from jax.experimental.pallas import tpu_sc as plsc

N = 10000
E = 320000
D = 128
DH = D // 2
R = 16
NC = 2
NS = 16
L = 16
NW = NC * NS
EPT = E // NS
EPW = E // NW
C = 80
NCH = EPT // C
PAIRS = NCH // 2
NPAD = 10240
RPW = NPAD // NS
EPTP = 20480

_mesh = lambda: plsc.VectorSubcoreMesh(
    core_axis_name="c", subcore_axis_name="s", num_cores=NC, num_subcores=NS)


def _zero_vmem_rows(buf, nrows, ncols):
  def zb(i, carry):
    for j in range(ncols // L):
      buf[i, pl.ds(j * L, L)] = jnp.zeros((L,), jnp.float32)
    return carry
  lax.fori_loop(0, nrows, zb, 0)


def _zero_vmem_1d(buf, n):
  def zd(i, carry):
    buf[pl.ds(i * L, L)] = jnp.zeros((L,), jnp.float32)
    return carry
  lax.fori_loop(0, n // L, zd, 0)


def _pass_call(mode, xh, gidx, sidx, *extra):
  first = mode == "first"
  attn = mode == "attn"

  if first:
    out_type = (
        jax.ShapeDtypeStruct((NC, NPAD, DH), jnp.float32),
        jax.ShapeDtypeStruct((NPAD,), jnp.float32),
        jax.ShapeDtypeStruct((E,), jnp.float32),
    )
  else:
    out_type = jax.ShapeDtypeStruct((NC, NPAD, DH), jnp.float32)

  scratch = [
      pltpu.VMEM((NCH, C), jnp.int32),
      pltpu.VMEM((EPTP,), jnp.float32),
      pltpu.VMEM((C, DH), jnp.float32),
      pltpu.VMEM((C, DH), jnp.float32),
      pltpu.VMEM((C, DH), jnp.float32),
      pltpu.VMEM((C, DH), jnp.float32),
      pltpu.VMEM((C,), jnp.int32),
      pltpu.VMEM_SHARED((NPAD, DH), jnp.float32),
      pltpu.SemaphoreType.DMA,
      pltpu.SemaphoreType.DMA,
  ]
  if first:
    scratch += [
        pltpu.VMEM((EPT,), jnp.int32),
        pltpu.VMEM((D,), jnp.float32),
        pltpu.VMEM_SHARED((NPAD,), jnp.float32),
    ]
  if attn:
    scratch += [pltpu.VMEM((NW, L), jnp.float32)]

  @functools.partial(
      pl.kernel,
      out_type=out_type,
      mesh=_mesh(),
      compiler_params=pltpu.CompilerParams(needs_layout_passes=False, use_tc_tiling_on_sc=False),
      scratch_types=scratch,
  )
  def k(*refs):
    if first:
      xh_hbm, g3, s_flat, aux, wtab_hbm = refs[:5]
      aggp, degp, weo = refs[5:8]
      rest = refs[8:]
    elif attn:
      xh_hbm, g3, s_flat, aux, sm_hbm = refs[:5]
      aggp = refs[5]
      rest = refs[6:]
    else:
      xh_hbm, g3, s_flat, aux = refs[:4]
      aggp = refs[4]
      rest = refs[5:]
    (s2, wloc, rin0, rin1, rout0, rout1, idx_d, agg_sh, gs0, gs1) = rest[:10]
    rest = rest[10:]
    if first:
      etloc, wtabv, deg_sh = rest[:3]
      rest = rest[3:]
    if attn:
      smv = rest[0]
    rin = (rin0, rin1)
    rout = (rout0, rout1)
    gsem = (gs0, gs1)

    c = lax.axis_index("c")
    s = lax.axis_index("s")
    base = s * EPT
    xcore = xh_hbm.at[c]

    _zero_vmem_rows(rout0, C, DH)
    for kk in range(RPW // C):
      pltpu.sync_copy(rout0, agg_sh.at[pl.ds(s * RPW + kk * C, C)])
    pltpu.sync_copy(g3.at[s], s2)
    if first:
      _zero_vmem_1d(wloc, RPW)
      pltpu.sync_copy(wloc.at[pl.ds(0, RPW)], deg_sh.at[pl.ds(s * RPW, RPW)])
      pltpu.sync_copy(aux.at[pl.ds(base, EPT)], etloc)
      pltpu.sync_copy(wtab_hbm, wtabv)

      def wg(kk, cy):
        ev = etloc[pl.ds(kk * L, L)]
        wloc[pl.ds(kk * L, L)] = plsc.load_gather(wtabv, [ev])
        return cy
      lax.fori_loop(0, EPT // L, wg, 0)

      @pl.when(c == 0)
      def _():
        pltpu.sync_copy(wloc.at[pl.ds(0, EPT)], weo.at[pl.ds(base, EPT)])
    elif attn:
      pltpu.sync_copy(sm_hbm, smv)
      acc = smv[0, :]
      for w in range(1, NW):
        acc = acc + smv[w, :]
      total = jnp.sum(acc)
      inv16 = jnp.ones((L,), jnp.float32) / (
          jnp.full((L,), total, jnp.float32) + jnp.float32(1e-9))
      pltpu.sync_copy(aux.at[pl.ds(base, EPT)], wloc.at[pl.ds(0, EPT)])

      def nr(kk, cy):
        wloc[pl.ds(kk * L, L)] = wloc[pl.ds(kk * L, L)] * inv16
        return cy
      lax.fori_loop(0, EPT // L, nr, 0)
    else:
      pltpu.sync_copy(aux.at[pl.ds(base, EPT)], wloc.at[pl.ds(0, EPT)])
    plsc.subcore_barrier()

    def scale_chunk(g, b):
      woff = g * C

      def sc(i, cy):
        w16 = plsc.load_gather(wloc, [jnp.full((L,), woff + i, jnp.int32)])
        for j in range(DH // L):
          rout[b][i, pl.ds(j * L, L)] = rin[b][i, pl.ds(j * L, L)] * w16
        return cy
      lax.fori_loop(0, C, sc, 0)

    pltpu.async_copy(xcore.at[s2.at[0]], rin0, gs0)
    pltpu.async_copy(xcore.at[s2.at[1]], rin1, gs1)

    def pair(p, cy):
      for b in range(2):
        g = p * 2 + b
        pltpu.make_async_copy(xcore.at[s2.at[g]], rin[b], gsem[b]).wait()
        scale_chunk(g, b)

        @pl.when(g + 2 < NCH)
        def _():
          pltpu.async_copy(xcore.at[s2.at[g + 2]], rin[b], gsem[b])

        pltpu.sync_copy(s_flat.at[pl.ds(base + g * C, C)], idx_d)
        pltpu.sync_copy(rout[b], agg_sh.at[idx_d], add=True)
        if first:
          @pl.when(c == 0)
          def _():
            pltpu.sync_copy(
                wloc.at[pl.ds(g * C, C)], deg_sh.at[idx_d], add=True)
      return cy

    lax.fori_loop(0, PAIRS, pair, 0)

    plsc.subcore_barrier()
    pltpu.sync_copy(agg_sh.at[pl.ds(s * RPW, RPW)],
                    aggp.at[c, pl.ds(s * RPW, RPW)])
    if first:
      @pl.when(c == 0)
      def _():
        pltpu.sync_copy(deg_sh.at[pl.ds(s * RPW, RPW)],
                        degp.at[pl.ds(s * RPW, RPW)])

  return k(xh, gidx, sidx, *extra)


def _edges_first(xh, src3, dst, et, wtab):
  return _pass_call("first", xh, src3, dst, et, wtab)


def _edges_next(xh, src3, dst, we):
  return _pass_call("next", xh, src3, dst, we)


def _attn_scatter(xh, dst3, src, ex, sm):
  return _pass_call("attn", xh, dst3, src, ex, sm)


def _attn_logits(sa, sb, src, dst, cvec):
  @functools.partial(
      pl.kernel,
      out_type=(
          jax.ShapeDtypeStruct((E,), jnp.float32),
          jax.ShapeDtypeStruct((NW, L), jnp.float32),
      ),
      mesh=_mesh(),
      compiler_params=pltpu.CompilerParams(needs_layout_passes=False, use_tc_tiling_on_sc=False),
      scratch_types=[
          pltpu.VMEM((NPAD,), jnp.float32),
          pltpu.VMEM((NPAD,), jnp.float32),
          pltpu.VMEM((EPW,), jnp.int32),
          pltpu.VMEM((EPW,), jnp.int32),
          pltpu.VMEM((EPW,), jnp.float32),
          pltpu.VMEM((L,), jnp.float32),
          pltpu.VMEM((L,), jnp.float32),
      ],
  )
  def k(sa_hbm, sb_hbm, src_hbm, dst_hbm, c_hbm, e_out, mx_out,
        sav, sbv, srcv, dstv, ev, mv, cv):
    c = lax.axis_index("c")
    s = lax.axis_index("s")
    wid = c * NS + s
    base = wid * EPW
    pltpu.sync_copy(sa_hbm, sav.at[pl.ds(0, N)])
    pltpu.sync_copy(sb_hbm, sbv.at[pl.ds(0, N)])
    pltpu.sync_copy(src_hbm.at[pl.ds(base, EPW)], srcv)
    pltpu.sync_copy(dst_hbm.at[pl.ds(base, EPW)], dstv)
    pltpu.sync_copy(c_hbm, cv)
    c16 = cv[...]

    def step(kk, m):
      sv = srcv[pl.ds(kk * L, L)]
      dv = dstv[pl.ds(kk * L, L)]
      a16 = plsc.load_gather(sav, [sv])
      b16 = plsc.load_gather(sbv, [dv])
      e16 = a16 + b16 + c16
      e16 = jnp.where(e16 >= 0.0, e16, e16 * jnp.float32(0.01))
      ev[pl.ds(kk * L, L)] = e16
      return jnp.maximum(m, e16)

    m = lax.fori_loop(0, EPW // L, step,
                      jnp.full((L,), -jnp.inf, jnp.float32))
    mv[...] = m
    pltpu.sync_copy(ev, e_out.at[pl.ds(base, EPW)])
    pltpu.sync_copy(mv, mx_out.at[wid])

  return k(sa, sb, src, dst, cvec)


def _attn_exp(e, mx):
  @functools.partial(
      pl.kernel,
      out_type=(
          jax.ShapeDtypeStruct((E,), jnp.float32),
          jax.ShapeDtypeStruct((NW, L), jnp.float32),
      ),
      mesh=_mesh(),
      compiler_params=pltpu.CompilerParams(needs_layout_passes=False, use_tc_tiling_on_sc=False),
      scratch_types=[
          pltpu.VMEM((EPW,), jnp.float32),
          pltpu.VMEM((NW, L), jnp.float32),
          pltpu.VMEM((L,), jnp.float32),
      ],
  )
  def k(e_hbm, mx_hbm, ex_out, sm_out, ev, mxv, sv):
    c = lax.axis_index("c")
    s = lax.axis_index("s")
    wid = c * NS + s
    base = wid * EPW
    pltpu.sync_copy(mx_hbm, mxv)
    m = mxv[0, :]
    for w in range(1, NW):
      m = jnp.maximum(m, mxv[w, :])
    gm = jnp.max(m)
    pltpu.sync_copy(e_hbm.at[pl.ds(base, EPW)], ev)

    def step(kk, acc):
      x16 = jnp.exp(ev[pl.ds(kk * L, L)] - gm)
      ev[pl.ds(kk * L, L)] = x16
      return acc + x16

    acc = lax.fori_loop(0, EPW // L, step, jnp.zeros((L,), jnp.float32))
    sv[...] = acc
    pltpu.sync_copy(ev, ex_out.at[pl.ds(base, EPW)])
    pltpu.sync_copy(sv, sm_out.at[wid])

  return k(e, mx)


BR = 80
NBLK = N // BR


def _prep_tc(raw, u3, en_w, re_w, rsa):
  def body(raw_ref, u3_ref, en_ref, rew_ref, rsa_ref,
           wtab_ref, av_ref, bv_ref, cv_ref):
    wtab_ref[...] = jax.nn.sigmoid(raw_ref[...])
    u0 = u3_ref[0, :][None, :]
    u1 = u3_ref[1, :]
    u2 = u3_ref[2, :][None, :]
    av_ref[...] = jnp.dot(u0, en_ref[...], preferred_element_type=jnp.float32)
    bv_ref[...] = jnp.dot(u2, en_ref[...], preferred_element_type=jnp.float32)
    r_term = jnp.sum(rew_ref[...] * rsa_ref[...], axis=1)
    cval = jnp.sum(r_term * u1)
    cv_ref[...] = jnp.full((1, L), cval, jnp.float32)

  return pl.pallas_call(
      body,
      out_shape=(
          jax.ShapeDtypeStruct((1, R), jnp.float32),
          jax.ShapeDtypeStruct((1, D), jnp.float32),
          jax.ShapeDtypeStruct((1, D), jnp.float32),
          jax.ShapeDtypeStruct((1, L), jnp.float32),
      ),
  )(raw, u3, en_w, re_w, rsa)


def _layer_tc(xh, aggp, degp, W):
  def body(x_ref, agg_ref, deg_ref, w_ref, o_ref):
    x = jnp.concatenate([x_ref[0], x_ref[1]], axis=1)
    agg = jnp.concatenate([agg_ref[0], agg_ref[1]], axis=1)
    deg = deg_ref[...] + jnp.float32(1e-6)
    h = agg / deg + x
    y = jnp.maximum(
        jnp.dot(h, w_ref[...], preferred_element_type=jnp.float32), 0.0)
    o_ref[0, :, :] = y[:, :DH]
    o_ref[1, :, :] = y[:, DH:]

  return pl.pallas_call(
      body,
      grid=(NBLK,),
      in_specs=[
          pl.BlockSpec((NC, BR, DH), lambda g: (0, g, 0)),
          pl.BlockSpec((NC, BR, DH), lambda g: (0, g, 0)),
          pl.BlockSpec((BR, 1), lambda g: (g, 0)),
          pl.BlockSpec((D, D), lambda g: (0, 0)),
      ],
      out_specs=pl.BlockSpec((NC, BR, DH), lambda g: (0, g, 0)),
      out_shape=jax.ShapeDtypeStruct((NC, N, DH), jnp.float32),
  )(xh, aggp, degp, W)


def _sab_tc(xh, ab):
  def body(x_ref, ab_ref, o_ref):
    x = jnp.concatenate([x_ref[0], x_ref[1]], axis=1)
    o_ref[...] = jnp.dot(x, ab_ref[...], preferred_element_type=jnp.float32)

  return pl.pallas_call(
      body,
      grid=(NBLK,),
      in_specs=[
          pl.BlockSpec((NC, BR, DH), lambda g: (0, g, 0)),
          pl.BlockSpec((D, 2), lambda g: (0, 0)),
      ],
      out_specs=pl.BlockSpec((BR, 2), lambda g: (g, 0)),
      out_shape=jax.ShapeDtypeStruct((N, 2), jnp.float32),
  )(xh, ab)


def _combine_tc(outp):
  def body(p_ref, o_ref):
    o_ref[...] = jnp.concatenate([p_ref[0], p_ref[1]], axis=1)

  return pl.pallas_call(
      body,
      grid=(NBLK,),
      in_specs=[pl.BlockSpec((NC, BR, DH), lambda g: (0, g, 0))],
      out_specs=pl.BlockSpec((BR, D), lambda g: (g, 0)),
      out_shape=jax.ShapeDtypeStruct((N, D), jnp.float32),
  )(outp)


def kernel(emb_e, W1, W2, W3, re_attention_weight, u, en_weight, re_weight,
           re_specific_attention, edge_index, edge_type):
  src = edge_index[0].astype(jnp.int32)
  dst = edge_index[1].astype(jnp.int32)
  et = edge_type.astype(jnp.int32)
  src3 = src.reshape(NS, NCH, C)
  dst3 = dst.reshape(NS, NCH, C)

  raw = re_attention_weight.reshape(1, R)
  u3 = u.reshape(3, D)
  rsa = re_specific_attention.reshape(1, D)

  wtab2, av, bv, cv = _prep_tc(raw, u3, en_weight, re_weight, rsa)
  wtab = jnp.pad(wtab2.reshape(R), (0, D - R))
  cvec = cv.reshape(L)

  x0h = jnp.stack([emb_e[:, :DH], emb_e[:, DH:]])

  aggp, deg, we = _edges_first(x0h, src3, dst, et, wtab)
  deg2 = deg[:N].reshape(N, 1)
  x1h = _layer_tc(x0h, aggp, deg2, W1)

  aggp2 = _edges_next(x1h, src3, dst, we)
  x2h = _layer_tc(x1h, aggp2, deg2, W2)

  aggp3 = _edges_next(x2h, src3, dst, we)
  x3h = _layer_tc(x2h, aggp3, deg2, W3)

  ab = jnp.stack([av[0], bv[0]], axis=1)
  sab = _sab_tc(x3h, ab)
  sa = sab[:, 0]
  sb = sab[:, 1]

  e, mx = _attn_logits(sa, sb, src, dst, cvec)
  ex, sm = _attn_exp(e, mx)
  outp = _attn_scatter(x3h, dst3, src, ex, sm)
  return _combine_tc(outp)
